# baseline (speedup 1.0000x reference)
import functools

import jax
import jax.numpy as jnp
from jax import lax
from jax.experimental import pallas as pl
from jax.experimental.pallas import tpu as pltpu
from jax.experimental.pallas import tpu_sc as plsc

_TOKENS = 16384
_HIDDEN = 4096
_EXPERTS = 64
_TOPK = 8
_BT = 1024
_NCHUNK = 1

_NC = 2
_NS = 16
_L = 16
_NW = _NC * _NS
_TPW = _TOKENS // _NW


def _logits_body(x_ref, w_ref, o_ref):
    o_ref[...] = jnp.dot(x_ref[...], w_ref[...],
                         preferred_element_type=jnp.float32)


def _compute_logits(hs, w_t, chunk, ct):
    nb = ct // _BT
    return pl.pallas_call(
        _logits_body,
        grid=(nb,),
        in_specs=[
            pl.BlockSpec((_BT, _HIDDEN), lambda i, c=chunk, n=nb: (c * n + i, 0)),
            pl.BlockSpec((_HIDDEN, _EXPERTS), lambda i: (0, 0)),
        ],
        out_specs=pl.BlockSpec((_BT, _EXPERTS), lambda i: (i, 0)),
        out_shape=jax.ShapeDtypeStruct((ct, _EXPERTS), jnp.float32),
    )(hs, w_t)


def _topk_body(tpw, logits_hbm, scores_hbm, idx_hbm, slab, sc_v, ix_v):
    wid = lax.axis_index("s") * _NC + lax.axis_index("c")
    base = wid * tpw
    pltpu.sync_copy(logits_hbm.at[pl.ds(base, tpw)], slab)

    lane = lax.iota(jnp.int32, _L)
    topmask = lane < _TOPK

    @plsc.parallel_loop(0, tpw, 1, unroll=4)
    def body(i):
        v0 = slab[i, pl.ds(0, _L)]
        v1 = slab[i, pl.ds(_L, _L)]
        v2 = slab[i, pl.ds(2 * _L, _L)]
        v3 = slab[i, pl.ds(3 * _L, _L)]
        s0k, s0i = plsc.sort_key_val(v0, lane, descending=True)
        s1k, s1i = plsc.sort_key_val(v1, lane + _L, descending=False)
        s2k, s2i = plsc.sort_key_val(v2, lane + 2 * _L, descending=True)
        s3k, s3i = plsc.sort_key_val(v3, lane + 3 * _L, descending=False)
        m = s0k >= s1k
        l01k, l01i = plsc.sort_key_val(jnp.where(m, s0k, s1k),
                                       jnp.where(m, s0i, s1i),
                                       descending=True)
        m = s2k >= s3k
        l23k, l23i = plsc.sort_key_val(jnp.where(m, s2k, s3k),
                                       jnp.where(m, s2i, s3i),
                                       descending=False)
        m = l01k >= l23k
        fk, fi = plsc.sort_key_val(jnp.where(m, l01k, l23k),
                                   jnp.where(m, l01i, l23i),
                                   descending=True)
        mx = jnp.max(fk)
        e = jnp.where(topmask, jnp.exp(fk - mx), 0.0)
        s = jnp.sum(e)
        plsc.store_compressed(sc_v.at[pl.ds(i * _TOPK, _L)], e / s,
                              mask=topmask)
        plsc.store_compressed(ix_v.at[pl.ds(i * _TOPK, _L)], fi,
                              mask=topmask)

    flat = tpw * _TOPK
    pltpu.sync_copy(sc_v.at[pl.ds(0, flat)],
                    scores_hbm.at[pl.ds(base * _TOPK, flat)])
    pltpu.sync_copy(ix_v.at[pl.ds(0, flat)],
                    idx_hbm.at[pl.ds(base * _TOPK, flat)])


def _router_topk(logits, ct):
    tpw = ct // _NW
    mesh = plsc.VectorSubcoreMesh(core_axis_name="c", subcore_axis_name="s",
                                  num_cores=_NC, num_subcores=_NS)
    fn = pl.kernel(
        functools.partial(_topk_body, tpw),
        out_type=(
            jax.ShapeDtypeStruct((ct * _TOPK,), jnp.float32),
            jax.ShapeDtypeStruct((ct * _TOPK,), jnp.int32),
        ),
        mesh=mesh,
        compiler_params=pltpu.CompilerParams(needs_layout_passes=False),
        scratch_types=[
            pltpu.VMEM((tpw, _EXPERTS), jnp.float32),
            pltpu.VMEM((tpw * _TOPK + _L,), jnp.float32),
            pltpu.VMEM((tpw * _TOPK + _L,), jnp.int32),
        ],
    )
    return fn(logits)


def kernel(hidden_states, weight):
    hs = hidden_states.reshape(-1, _HIDDEN)
    w_t = weight.T
    ct = _TOKENS // _NCHUNK
    parts = []
    for c in range(_NCHUNK):
        lg = _compute_logits(hs, w_t, c, ct)
        sc, ix = _router_topk(lg, ct)
        parts.append((lg, sc, ix))
    logits = jnp.concatenate([p[0] for p in parts], axis=0)
    scores = jnp.concatenate([p[1] for p in parts]).reshape(_TOKENS, _TOPK)
    idx = jnp.concatenate([p[2] for p in parts]).reshape(_TOKENS, _TOPK)
    return (logits, scores, idx)


# speedup vs baseline: 1.0055x; 1.0055x over previous
import functools

import jax
import jax.numpy as jnp
from jax import lax
from jax.experimental import pallas as pl
from jax.experimental.pallas import tpu as pltpu
from jax.experimental.pallas import tpu_sc as plsc

_TOKENS = 16384
_HIDDEN = 4096
_EXPERTS = 64
_TOPK = 8
_BT = 512
_NCHUNK = 1

_NC = 2
_NS = 16
_L = 16
_NW = _NC * _NS
_TPW = _TOKENS // _NW


def _logits_body(x_ref, w_ref, o_ref):
    o_ref[...] = jnp.dot(x_ref[...], w_ref[...],
                         preferred_element_type=jnp.float32)


def _compute_logits(hs, w_t, chunk, ct):
    nb = ct // _BT
    return pl.pallas_call(
        _logits_body,
        grid=(nb,),
        in_specs=[
            pl.BlockSpec((_BT, _HIDDEN), lambda i, c=chunk, n=nb: (c * n + i, 0)),
            pl.BlockSpec((_HIDDEN, _EXPERTS), lambda i: (0, 0)),
        ],
        out_specs=pl.BlockSpec((_BT, _EXPERTS), lambda i: (i, 0)),
        out_shape=jax.ShapeDtypeStruct((ct, _EXPERTS), jnp.float32),
    )(hs, w_t)


def _topk_body(tpw, logits_hbm, scores_hbm, idx_hbm, slab, sc_v, ix_v):
    wid = lax.axis_index("s") * _NC + lax.axis_index("c")
    base = wid * tpw
    pltpu.sync_copy(logits_hbm.at[pl.ds(base, tpw)], slab)

    lane = lax.iota(jnp.int32, _L)
    topmask = lane < _TOPK

    @plsc.parallel_loop(0, tpw, 1, unroll=8)
    def body(i):
        v0 = slab[i, pl.ds(0, _L)]
        v1 = slab[i, pl.ds(_L, _L)]
        v2 = slab[i, pl.ds(2 * _L, _L)]
        v3 = slab[i, pl.ds(3 * _L, _L)]
        s0k, s0i = plsc.sort_key_val(v0, lane, descending=True)
        s1k, s1i = plsc.sort_key_val(v1, lane + _L, descending=False)
        s2k, s2i = plsc.sort_key_val(v2, lane + 2 * _L, descending=True)
        s3k, s3i = plsc.sort_key_val(v3, lane + 3 * _L, descending=False)
        m = s0k >= s1k
        l01k, l01i = plsc.sort_key_val(jnp.where(m, s0k, s1k),
                                       jnp.where(m, s0i, s1i),
                                       descending=True)
        m = s2k >= s3k
        l23k, l23i = plsc.sort_key_val(jnp.where(m, s2k, s3k),
                                       jnp.where(m, s2i, s3i),
                                       descending=False)
        m = l01k >= l23k
        fk, fi = plsc.sort_key_val(jnp.where(m, l01k, l23k),
                                   jnp.where(m, l01i, l23i),
                                   descending=True)
        mx = jnp.max(fk)
        e = jnp.where(topmask, jnp.exp(fk - mx), 0.0)
        s = jnp.sum(e)
        plsc.store_compressed(sc_v.at[pl.ds(i * _TOPK, _L)], e / s,
                              mask=topmask)
        plsc.store_compressed(ix_v.at[pl.ds(i * _TOPK, _L)], fi,
                              mask=topmask)

    flat = tpw * _TOPK
    pltpu.sync_copy(sc_v.at[pl.ds(0, flat)],
                    scores_hbm.at[pl.ds(base * _TOPK, flat)])
    pltpu.sync_copy(ix_v.at[pl.ds(0, flat)],
                    idx_hbm.at[pl.ds(base * _TOPK, flat)])


def _router_topk(logits, ct):
    tpw = ct // _NW
    mesh = plsc.VectorSubcoreMesh(core_axis_name="c", subcore_axis_name="s",
                                  num_cores=_NC, num_subcores=_NS)
    fn = pl.kernel(
        functools.partial(_topk_body, tpw),
        out_type=(
            jax.ShapeDtypeStruct((ct * _TOPK,), jnp.float32),
            jax.ShapeDtypeStruct((ct * _TOPK,), jnp.int32),
        ),
        mesh=mesh,
        compiler_params=pltpu.CompilerParams(needs_layout_passes=False),
        scratch_types=[
            pltpu.VMEM((tpw, _EXPERTS), jnp.float32),
            pltpu.VMEM((tpw * _TOPK + _L,), jnp.float32),
            pltpu.VMEM((tpw * _TOPK + _L,), jnp.int32),
        ],
    )
    return fn(logits)


def kernel(hidden_states, weight):
    hs = hidden_states.reshape(-1, _HIDDEN)
    w_t = weight.T
    ct = _TOKENS // _NCHUNK
    parts = []
    for c in range(_NCHUNK):
        lg = _compute_logits(hs, w_t, c, ct)
        sc, ix = _router_topk(lg, ct)
        parts.append((lg, sc, ix))
    logits = jnp.concatenate([p[0] for p in parts], axis=0)
    scores = jnp.concatenate([p[1] for p in parts]).reshape(_TOKENS, _TOPK)
    idx = jnp.concatenate([p[2] for p in parts]).reshape(_TOKENS, _TOPK)
    return (logits, scores, idx)


# speedup vs baseline: 1.0132x; 1.0076x over previous
import functools

import jax
import jax.numpy as jnp
from jax import lax
from jax.experimental import pallas as pl
from jax.experimental.pallas import tpu as pltpu
from jax.experimental.pallas import tpu_sc as plsc

_TOKENS = 16384
_HIDDEN = 4096
_EXPERTS = 64
_TOPK = 8
_BT = 512
_NCHUNK = 1

_NC = 2
_NS = 16
_L = 16
_NW = _NC * _NS
_TPW = _TOKENS // _NW


def _logits_body(x_ref, w_ref, o_ref):
    o_ref[...] = jnp.dot(x_ref[...], w_ref[...],
                         preferred_element_type=jnp.float32)


def _compute_logits(hs, w_t, chunk, ct):
    nb = ct // _BT
    return pl.pallas_call(
        _logits_body,
        grid=(nb,),
        in_specs=[
            pl.BlockSpec((_BT, _HIDDEN), lambda i, c=chunk, n=nb: (c * n + i, 0)),
            pl.BlockSpec((_HIDDEN, _EXPERTS), lambda i: (0, 0)),
        ],
        out_specs=pl.BlockSpec((_BT, _EXPERTS), lambda i: (i, 0)),
        out_shape=jax.ShapeDtypeStruct((ct, _EXPERTS), jnp.float32),
    )(hs, w_t)


def _topk_body(tpw, logits_hbm, scores_hbm, idx_hbm, slab, sc_v, ix_v):
    wid = lax.axis_index("s") * _NC + lax.axis_index("c")
    base = wid * tpw
    pltpu.sync_copy(logits_hbm.at[pl.ds(base, tpw)], slab)

    lane = lax.iota(jnp.int32, _L)
    topmask = lane < _TOPK

    @plsc.parallel_loop(0, tpw, 1, unroll=4)
    def body(i):
        v0 = slab[i, pl.ds(0, _L)]
        v1 = slab[i, pl.ds(_L, _L)]
        v2 = slab[i, pl.ds(2 * _L, _L)]
        v3 = slab[i, pl.ds(3 * _L, _L)]
        s0k, s0i = plsc.sort_key_val(v0, lane, descending=True)
        s1k, s1i = plsc.sort_key_val(v1, lane + _L, descending=False)
        s2k, s2i = plsc.sort_key_val(v2, lane + 2 * _L, descending=True)
        s3k, s3i = plsc.sort_key_val(v3, lane + 3 * _L, descending=False)
        m = s0k >= s1k
        l01k, l01i = plsc.sort_key_val(jnp.where(m, s0k, s1k),
                                       jnp.where(m, s0i, s1i),
                                       descending=True)
        m = s2k >= s3k
        l23k, l23i = plsc.sort_key_val(jnp.where(m, s2k, s3k),
                                       jnp.where(m, s2i, s3i),
                                       descending=False)
        m = l01k >= l23k
        fk, fi = plsc.sort_key_val(jnp.where(m, l01k, l23k),
                                   jnp.where(m, l01i, l23i),
                                   descending=True)
        mx = jnp.max(fk)
        e = jnp.where(topmask, jnp.exp(fk - mx), 0.0)
        s = jnp.sum(e)
        plsc.store_compressed(sc_v.at[pl.ds(i * _TOPK, _L)], e / s,
                              mask=topmask)
        plsc.store_compressed(ix_v.at[pl.ds(i * _TOPK, _L)], fi,
                              mask=topmask)

    flat = tpw * _TOPK
    pltpu.sync_copy(sc_v.at[pl.ds(0, flat)],
                    scores_hbm.at[pl.ds(base * _TOPK, flat)])
    pltpu.sync_copy(ix_v.at[pl.ds(0, flat)],
                    idx_hbm.at[pl.ds(base * _TOPK, flat)])


def _router_topk(logits, ct):
    tpw = ct // _NW
    mesh = plsc.VectorSubcoreMesh(core_axis_name="c", subcore_axis_name="s",
                                  num_cores=_NC, num_subcores=_NS)
    fn = pl.kernel(
        functools.partial(_topk_body, tpw),
        out_type=(
            jax.ShapeDtypeStruct((ct * _TOPK,), jnp.float32),
            jax.ShapeDtypeStruct((ct * _TOPK,), jnp.int32),
        ),
        mesh=mesh,
        compiler_params=pltpu.CompilerParams(needs_layout_passes=False),
        scratch_types=[
            pltpu.VMEM((tpw, _EXPERTS), jnp.float32),
            pltpu.VMEM((tpw * _TOPK + _L,), jnp.float32),
            pltpu.VMEM((tpw * _TOPK + _L,), jnp.int32),
        ],
    )
    return fn(logits)


def kernel(hidden_states, weight):
    hs = hidden_states.reshape(-1, _HIDDEN)
    w_t = weight.T
    ct = _TOKENS // _NCHUNK
    parts = []
    for c in range(_NCHUNK):
        lg = _compute_logits(hs, w_t, c, ct)
        sc, ix = _router_topk(lg, ct)
        parts.append((lg, sc, ix))
    logits = jnp.concatenate([p[0] for p in parts], axis=0)
    scores = jnp.concatenate([p[1] for p in parts]).reshape(_TOKENS, _TOPK)
    idx = jnp.concatenate([p[2] for p in parts]).reshape(_TOKENS, _TOPK)
    return (logits, scores, idx)


# speedup vs baseline: 1.0389x; 1.0254x over previous
import functools

import jax
import jax.numpy as jnp
from jax import lax
from jax.experimental import pallas as pl
from jax.experimental.pallas import tpu as pltpu
from jax.experimental.pallas import tpu_sc as plsc

_TOKENS = 16384
_HIDDEN = 4096
_EXPERTS = 64
_TOPK = 8
_BT = 512
_NCHUNK = 1

_NC = 2
_NS = 16
_L = 16
_NW = _NC * _NS
_TPW = _TOKENS // _NW


def _logits_body(x_ref, w_ref, o_ref):
    o_ref[...] = lax.dot_general(
        x_ref[...], w_ref[...],
        dimension_numbers=(((1,), (1,)), ((), ())),
        preferred_element_type=jnp.float32)


def _compute_logits(hs, w, chunk, ct):
    nb = ct // _BT
    return pl.pallas_call(
        _logits_body,
        grid=(nb,),
        in_specs=[
            pl.BlockSpec((_BT, _HIDDEN), lambda i, c=chunk, n=nb: (c * n + i, 0)),
            pl.BlockSpec((_EXPERTS, _HIDDEN), lambda i: (0, 0)),
        ],
        out_specs=pl.BlockSpec((_BT, _EXPERTS), lambda i: (i, 0)),
        out_shape=jax.ShapeDtypeStruct((ct, _EXPERTS), jnp.float32),
    )(hs, w)


def _topk_body(tpw, logits_hbm, scores_hbm, idx_hbm, slab, sc_v, ix_v):
    wid = lax.axis_index("s") * _NC + lax.axis_index("c")
    base = wid * tpw
    pltpu.sync_copy(logits_hbm.at[pl.ds(base, tpw)], slab)

    lane = lax.iota(jnp.int32, _L)
    topmask = lane < _TOPK

    @plsc.parallel_loop(0, tpw, 1, unroll=4)
    def body(i):
        v0 = slab[i, pl.ds(0, _L)]
        v1 = slab[i, pl.ds(_L, _L)]
        v2 = slab[i, pl.ds(2 * _L, _L)]
        v3 = slab[i, pl.ds(3 * _L, _L)]
        s0k, s0i = plsc.sort_key_val(v0, lane, descending=True)
        s1k, s1i = plsc.sort_key_val(v1, lane + _L, descending=False)
        s2k, s2i = plsc.sort_key_val(v2, lane + 2 * _L, descending=True)
        s3k, s3i = plsc.sort_key_val(v3, lane + 3 * _L, descending=False)
        m = s0k >= s1k
        l01k, l01i = plsc.sort_key_val(jnp.where(m, s0k, s1k),
                                       jnp.where(m, s0i, s1i),
                                       descending=True)
        m = s2k >= s3k
        l23k, l23i = plsc.sort_key_val(jnp.where(m, s2k, s3k),
                                       jnp.where(m, s2i, s3i),
                                       descending=False)
        m = l01k >= l23k
        fk, fi = plsc.sort_key_val(jnp.where(m, l01k, l23k),
                                   jnp.where(m, l01i, l23i),
                                   descending=True)
        mx = jnp.max(fk)
        e = jnp.where(topmask, jnp.exp(fk - mx), 0.0)
        s = jnp.sum(e)
        plsc.store_compressed(sc_v.at[pl.ds(i * _TOPK, _L)], e / s,
                              mask=topmask)
        plsc.store_compressed(ix_v.at[pl.ds(i * _TOPK, _L)], fi,
                              mask=topmask)

    flat = tpw * _TOPK
    pltpu.sync_copy(sc_v.at[pl.ds(0, flat)],
                    scores_hbm.at[pl.ds(base * _TOPK, flat)])
    pltpu.sync_copy(ix_v.at[pl.ds(0, flat)],
                    idx_hbm.at[pl.ds(base * _TOPK, flat)])


def _router_topk(logits, ct):
    tpw = ct // _NW
    mesh = plsc.VectorSubcoreMesh(core_axis_name="c", subcore_axis_name="s",
                                  num_cores=_NC, num_subcores=_NS)
    fn = pl.kernel(
        functools.partial(_topk_body, tpw),
        out_type=(
            jax.ShapeDtypeStruct((ct * _TOPK,), jnp.float32),
            jax.ShapeDtypeStruct((ct * _TOPK,), jnp.int32),
        ),
        mesh=mesh,
        compiler_params=pltpu.CompilerParams(needs_layout_passes=False),
        scratch_types=[
            pltpu.VMEM((tpw, _EXPERTS), jnp.float32),
            pltpu.VMEM((tpw * _TOPK + _L,), jnp.float32),
            pltpu.VMEM((tpw * _TOPK + _L,), jnp.int32),
        ],
    )
    return fn(logits)


def kernel(hidden_states, weight):
    hs = hidden_states.reshape(-1, _HIDDEN)
    ct = _TOKENS // _NCHUNK
    parts = []
    for c in range(_NCHUNK):
        lg = _compute_logits(hs, weight, c, ct)
        sc, ix = _router_topk(lg, ct)
        parts.append((lg, sc, ix))
    logits = jnp.concatenate([p[0] for p in parts], axis=0)
    scores = jnp.concatenate([p[1] for p in parts]).reshape(_TOKENS, _TOPK)
    idx = jnp.concatenate([p[2] for p in parts]).reshape(_TOKENS, _TOPK)
    return (logits, scores, idx)
